# baseline (speedup 1.0000x reference)
import functools

import jax
import jax.numpy as jnp
from jax import lax
from jax.experimental import pallas as pl
from jax.experimental.pallas import tpu as pltpu
from jax.experimental.pallas import tpu_sc as plsc

NFEAT = 56
DIM = 128
CBITS = 7
NCHUNK = 8
CROWS = 1 << CBITS
TROWS = NCHUNK * CROWS
NLANE = 16
NCORE = 2
NSUBCORE = 16


def _table_body(tab_ref, t_ref):
    r = lax.broadcasted_iota(jnp.int32, (TROWS, NFEAT), 0)
    i = lax.broadcasted_iota(jnp.int32, (TROWS, NFEAT), 1)
    c = r // CROWS
    b = r % CROWS
    j = i - c * CBITS
    inch = (j >= 0) & (j < CBITS)
    bit = jnp.right_shift(b, jnp.clip(j, 0, CBITS - 1)) & 1
    tab = tab_ref[...]
    m1 = (inch & (bit == 1)).astype(jnp.float32)
    m0 = (inch & (bit == 0)).astype(jnp.float32)
    dn = (((1,), (0,)), ((), ()))
    t_ref[...] = (
        lax.dot_general(m1, tab[:, 1, :], dn, preferred_element_type=jnp.float32)
        + lax.dot_general(m0, tab[:, 0, :], dn, preferred_element_type=jnp.float32))


@functools.lru_cache(maxsize=None)
def _make_sc(n):
    nw = NCORE * NSUBCORE
    rpw = n // nw
    assert n % nw == 0
    bn = 125
    assert rpw % bn == 0
    nblk = rpw // bn
    npair = bn // 2
    ngrp = (bn + NLANE - 1) // NLANE
    bnpad = ngrp * NLANE

    def body(x_hbm, t_hbm, out_hbm, x_v, t_v, w_v, o_v):
        wid = lax.axis_index("s") * NCORE + lax.axis_index("c")
        row0 = wid * rpw
        pltpu.sync_copy(t_hbm, t_v)
        iota = lax.iota(jnp.int32, NLANE)
        iota_x = iota * NFEAT
        iota_w = iota * NCHUNK

        hi_mask = jnp.int32(-65536)

        def unpack2(v):
            a = lax.bitcast_convert_type(lax.shift_left(v, 16), jnp.float32)
            b = lax.bitcast_convert_type(v & hi_mask, jnp.float32)
            return a, b

        def pack_group(g, carry):
            xbase = iota_x + g * (NLANE * NFEAT)
            wbase = iota_w + g * (NLANE * NCHUNK)
            for c in range(NCHUNK):
                code = None
                for j in range(CBITS):
                    col = plsc.load_gather(x_v, [xbase + (c * CBITS + j)])
                    term = col if j == 0 else lax.shift_left(col, j)
                    code = term if code is None else code + term
                plsc.store_scatter(w_v, [wbase + c], code + c * CROWS)
            return carry

        def accum_row(wvec, lane0, r):
            o0 = wvec[lane0] * (DIM // 2)
            accs = []
            for g in range(4):
                a, b = unpack2(t_v[pl.ds(o0 + g * NLANE, NLANE)])
                accs.append([a, b])
            for ci in range(1, NCHUNK):
                oc = wvec[lane0 + ci] * (DIM // 2)
                for g in range(4):
                    a, b = unpack2(t_v[pl.ds(oc + g * NLANE, NLANE)])
                    accs[g][0] += a
                    accs[g][1] += b
            for g in range(4):
                o_v[pl.ds(r * DIM + g * 32, NLANE)] = accs[g][0]
                o_v[pl.ds(r * DIM + g * 32 + NLANE, NLANE)] = accs[g][1]

        def block(blk, carry):
            r0 = row0 + blk * bn
            pltpu.sync_copy(x_hbm.at[pl.ds(r0 * NFEAT, bn * NFEAT)],
                            x_v.at[pl.ds(0, bn * NFEAT)])
            lax.fori_loop(0, ngrp, pack_group, 0)

            def pair(p2, carry2):
                wvec = w_v[pl.ds(p2 * NLANE, NLANE)]
                accum_row(wvec, 0, 2 * p2)
                accum_row(wvec, NCHUNK, 2 * p2 + 1)
                return carry2

            lax.fori_loop(0, npair, pair, 0)
            wtail = w_v[pl.ds((bn - 1) * NCHUNK, NLANE)]
            accum_row(wtail, 0, bn - 1)
            pltpu.sync_copy(o_v, out_hbm.at[pl.ds(r0 * DIM, bn * DIM)])
            return carry

        lax.fori_loop(0, nblk, block, 0)

    return pl.kernel(
        body,
        out_type=jax.ShapeDtypeStruct((n * DIM,), jnp.float32),
        mesh=plsc.VectorSubcoreMesh(core_axis_name="c", subcore_axis_name="s",
                                    num_cores=NCORE, num_subcores=NSUBCORE),
        compiler_params=pltpu.CompilerParams(needs_layout_passes=False),
        scratch_types=[
            pltpu.VMEM((bnpad * NFEAT,), jnp.int32),
            pltpu.VMEM((TROWS * DIM // 2,), jnp.int32),
            pltpu.VMEM((bnpad * NCHUNK,), jnp.int32),
            pltpu.VMEM((bn * DIM,), jnp.float32),
        ],
    )


def kernel(x, tables):
    n = x.shape[0]
    t = pl.pallas_call(
        _table_body,
        out_shape=jax.ShapeDtypeStruct((TROWS, DIM), jnp.float32),
    )(tables)
    tb = lax.bitcast_convert_type(
        t.reshape(TROWS, 4, 2, NLANE).astype(jnp.bfloat16),
        jnp.uint16).astype(jnp.uint32)
    t_pk = lax.bitcast_convert_type(
        tb[:, :, 0, :] | (tb[:, :, 1, :] << 16), jnp.int32)
    out = _make_sc(n)(x.reshape(-1), t_pk.reshape(-1))
    return out.reshape(n, DIM)
